# baseline (speedup 1.0000x reference)
import functools

import jax
import jax.numpy as jnp
from jax import lax
from jax.experimental import pallas as pl
from jax.experimental.pallas import tpu as pltpu
from jax.experimental.pallas import tpu_sc as plsc

_CHUNK = 16
_NBUF = 7


def kernel(input_ids, table):
    _, ll = input_ids.shape
    _, d = table.shape

    info = plsc.get_sparse_core_info()
    nw = info.num_cores * info.num_subcores
    rows_per_w = ll // nw
    nchunks = rows_per_w // _CHUNK

    mesh = plsc.VectorSubcoreMesh(core_axis_name="c", subcore_axis_name="s")

    scratch = [pltpu.VMEM((_CHUNK, d), table.dtype) for _ in range(_NBUF)]
    scratch += [pltpu.SemaphoreType.DMA, pltpu.SemaphoreType.DMA]

    @functools.partial(
        pl.kernel,
        mesh=mesh,
        out_type=jax.ShapeDtypeStruct((ll, d), table.dtype),
        scratch_types=scratch,
    )
    def copy_k(table_hbm, out_hbm, *rest):
        bufs = rest[:_NBUF]
        isem, osem = rest[_NBUF:]

        wid = lax.axis_index("s") * info.num_cores + lax.axis_index("c")
        base = wid * rows_per_w

        def start_in(i):
            return pltpu.async_copy(
                table_hbm.at[pl.ds(base + i * _CHUNK, _CHUNK)],
                bufs[i % _NBUF], isem)

        in_h = [None] * nchunks
        out_h = [None] * nchunks
        out_waited = [False] * nchunks

        for i in range(min(_NBUF, nchunks)):
            in_h[i] = start_in(i)
        for i in range(nchunks):
            in_h[i].wait()
            out_h[i] = pltpu.async_copy(
                bufs[i % _NBUF],
                out_hbm.at[pl.ds(base + i * _CHUNK, _CHUNK)], osem)
            j = i + _NBUF
            if j < nchunks:
                out_h[i].wait()
                out_waited[i] = True
                in_h[j] = start_in(j)
        for i in range(nchunks):
            if not out_waited[i]:
                out_h[i].wait()

    return copy_k(table)[None]


# speedup vs baseline: 1.0278x; 1.0278x over previous
import functools

import jax
import jax.numpy as jnp
from jax import lax
from jax.experimental import pallas as pl
from jax.experimental.pallas import tpu as pltpu
from jax.experimental.pallas import tpu_sc as plsc

_CHUNK = 16
_NBUF = 7


def kernel(input_ids, table):
    _, ll = input_ids.shape
    _, d = table.shape

    info = plsc.get_sparse_core_info()
    nw = info.num_cores * info.num_subcores
    rows_per_w = ll // nw
    nchunks = rows_per_w // _CHUNK

    mesh = plsc.VectorSubcoreMesh(core_axis_name="c", subcore_axis_name="s")

    scratch = [pltpu.VMEM((_CHUNK, d), table.dtype) for _ in range(_NBUF)]
    scratch += [pltpu.SemaphoreType.DMA for _ in range(2 * nchunks)]

    @functools.partial(
        pl.kernel,
        mesh=mesh,
        out_type=jax.ShapeDtypeStruct((ll, d), table.dtype),
        scratch_types=scratch,
    )
    def copy_k(table_hbm, out_hbm, *rest):
        bufs = rest[:_NBUF]
        isems = rest[_NBUF:_NBUF + nchunks]
        osems = rest[_NBUF + nchunks:]

        wid = lax.axis_index("s") * info.num_cores + lax.axis_index("c")
        base = wid * rows_per_w

        def start_in(i):
            return pltpu.async_copy(
                table_hbm.at[pl.ds(base + i * _CHUNK, _CHUNK)],
                bufs[i % _NBUF], isems[i])

        in_h = [None] * nchunks
        out_h = [None] * nchunks
        out_waited = [False] * nchunks

        for i in range(min(_NBUF, nchunks)):
            in_h[i] = start_in(i)
        for i in range(nchunks):
            in_h[i].wait()
            out_h[i] = pltpu.async_copy(
                bufs[i % _NBUF],
                out_hbm.at[pl.ds(base + i * _CHUNK, _CHUNK)], osems[i])
            j = i + _NBUF
            if j < nchunks:
                out_h[i].wait()
                out_waited[i] = True
                in_h[j] = start_in(j)
        for i in range(nchunks):
            if not out_waited[i]:
                out_h[i].wait()

    return copy_k(table)[None]
